# baseline (speedup 1.0000x reference)
import functools

import jax
import jax.numpy as jnp
from jax import lax
from jax.experimental import pallas as pl
from jax.experimental.pallas import tpu as pltpu
from jax.experimental.pallas import tpu_sc as plsc

_DIMS = [
    ("e1", 5, 6, 1, 2), ("n1", 5, 6, 8, 10),
    ("e2", 16, 32, 8, 16), ("n2", 16, 32, 48, 24),
    ("e3", 56, 24, 48, 24), ("n3", 56, 24, 48, 24),
    ("e4", 48, 13, 48, 13), ("n4", 48, 13, 26, 8),
    ("e5", 21, 3, 26, 3), ("n5", 21, 3, 6, 3),
]

_N = 2048
_E = 65536
_NC, _NS = 2, 16
_NW = _NC * _NS
_EPW = _E // _NW
_QC = 512
_NQ = _EPW // _QC

_mesh = plsc.VectorSubcoreMesh(
    core_axis_name="c", subcore_axis_name="s", num_cores=_NC, num_subcores=_NS)


def _pad8(w):
    return ((w + 7) // 8) * 8


def _pad_rows(m, rows):
    if m.shape[0] == rows:
        return m
    return jnp.concatenate(
        [m, jnp.zeros((rows - m.shape[0],) + m.shape[1:], m.dtype)], axis=0)


def _make_gather(ox, oxp):
    out_t = jax.ShapeDtypeStruct((_NW, oxp, _EPW), jnp.float32)
    scratch = [
        pltpu.VMEM((oxp * _N,), jnp.float32),
        pltpu.VMEM((_EPW,), jnp.int32),
        pltpu.VMEM((_EPW,), jnp.int32),
        [pltpu.VMEM((oxp, _QC), jnp.float32) for _ in range(2)],
        [pltpu.SemaphoreType.DMA for _ in range(2)],
    ]

    @functools.partial(
        pl.kernel, out_type=out_t, mesh=_mesh, scratch_types=scratch,
        name=f"sc_gather_{ox}",
        compiler_params=pltpu.CompilerParams(needs_layout_passes=False))
    def gk(p_hbm, src_hbm, dst_hbm, g_hbm, tbl_v, is_v, id_v, obufs, sems):
        wid = lax.axis_index("s") * _NC + lax.axis_index("c")
        ebase = wid * _EPW
        pltpu.sync_copy(p_hbm, tbl_v)
        pltpu.sync_copy(src_hbm.at[pl.ds(ebase, _EPW)], is_v)
        pltpu.sync_copy(dst_hbm.at[pl.ds(ebase, _EPW)], id_v)
        descs = [None, None]
        for q in range(_NQ):
            out_v = obufs[q % 2]
            if descs[q % 2] is not None:
                descs[q % 2].wait()

            def body(i, _):
                for u in range(4):
                    o = q * _QC + (i * 4 + u) * 16
                    i16 = is_v[pl.ds(o, 16)]
                    j16 = id_v[pl.ds(o, 16)]
                    for c in range(ox):
                        v = (plsc.load_gather(tbl_v, [i16 + c * _N])
                             + plsc.load_gather(tbl_v, [j16 + c * _N]))
                        out_v[c, pl.ds((i * 4 + u) * 16, 16)] = v
                return ()

            lax.fori_loop(0, _QC // 64, body, ())
            descs[q % 2] = pltpu.async_copy(
                out_v, g_hbm.at[wid, :, pl.ds(q * _QC, _QC)], sems[q % 2])
        for d in descs:
            d.wait()

    return gk


def _make_scatter(w, wp):
    out_t = jax.ShapeDtypeStruct((_NW, wp * _N), jnp.float32)
    scratch = [
        pltpu.VMEM((wp, _EPW), jnp.float32),
        pltpu.VMEM((_EPW,), jnp.int32),
        pltpu.VMEM((wp * _N,), jnp.float32),
    ]

    @functools.partial(
        pl.kernel, out_type=out_t, mesh=_mesh, scratch_types=scratch,
        name=f"sc_scatter_{w}",
        compiler_params=pltpu.CompilerParams(needs_layout_passes=False))
    def sk(s_hbm, src_hbm, zeros_hbm, agg_hbm, s_v, is_v, acc_v):
        wid = lax.axis_index("s") * _NC + lax.axis_index("c")
        ebase = wid * _EPW
        pltpu.sync_copy(s_hbm.at[wid], s_v)
        pltpu.sync_copy(src_hbm.at[pl.ds(ebase, _EPW)], is_v)
        pltpu.sync_copy(zeros_hbm, acc_v)

        def body(g, _):
            for u in range(4):
                o = (g * 4 + u) * 16
                i16 = is_v[pl.ds(o, 16)]
                for c in range(w):
                    plsc.addupdate_scatter(acc_v, [i16 + c * _N],
                                           s_v[c, pl.ds(o, 16)])
            return ()

        lax.fori_loop(0, _EPW // 64, body, ())
        pltpu.sync_copy(acc_v, agg_hbm.at[wid])

    return sk


def _dense_body(ox, _dense_w, gt_ref, h_ref, bxe_ref, wee_ref, bee_ref,
                wen_ref, hout_ref, s_ref):
    g = gt_ref[0][:ox] + bxe_ref[...]
    t = jnp.dot(wee_ref[...], h_ref[...],
                preferred_element_type=jnp.float32) + bee_ref[...]
    hk = jax.nn.relu(jnp.concatenate([g, t], axis=0))
    hout_ref[...] = hk
    s_ref[0] = jnp.dot(wen_ref[...], hk, preferred_element_type=jnp.float32)


def _make_dense(ew_prev, ox, oxp, oee, ewk, wp):
    const = lambda i: (0, 0)
    return pl.pallas_call(
        functools.partial(_dense_body, ox, wp),
        grid=(_NW,),
        in_specs=[
            pl.BlockSpec((1, oxp, _EPW), lambda i: (i, 0, 0)),
            pl.BlockSpec((ew_prev, _EPW), lambda i: (0, i)),
            pl.BlockSpec((ox, 1), const),
            pl.BlockSpec((oee, ew_prev), const),
            pl.BlockSpec((oee, 1), const),
            pl.BlockSpec((wp, ewk), const),
        ],
        out_specs=[
            pl.BlockSpec((ewk, _EPW), lambda i: (0, i)),
            pl.BlockSpec((1, wp, _EPW), lambda i: (i, 0, 0)),
        ],
        out_shape=[
            jax.ShapeDtypeStruct((ewk, _E), jnp.float32),
            jax.ShapeDtypeStruct((_NW, wp, _EPW), jnp.float32),
        ],
    )


def _node_body(oen, is_last, x_ref, agg_ref, wxn_ref, bxn_ref, ben_ref,
               wxe_ref, x_out, p_out=None):
    agg = jnp.sum(agg_ref[...], axis=0)[:oen]
    xs = jnp.dot(wxn_ref[...], x_ref[...],
                 preferred_element_type=jnp.float32) + bxn_ref[...]
    xn = jax.nn.relu(jnp.concatenate([xs, agg + ben_ref[...]], axis=0))
    if is_last:
        state = jnp.sum(xn, axis=1, keepdims=True)
        x_out[...] = jnp.concatenate(
            [jnp.broadcast_to(state, xn.shape), xn], axis=0)
    else:
        x_out[...] = xn
        p_out[...] = jnp.dot(wxe_ref[...], xn,
                             preferred_element_type=jnp.float32)


def _make_node(nw, oxn, oen, wp, nwn, oxp_next, is_last):
    const = lambda: (0, 0)
    out_w = 2 * nwn if is_last else nwn
    out_shape = [jax.ShapeDtypeStruct((out_w, _N), jnp.float32)]
    out_specs = [pl.BlockSpec((out_w, _N), const)]
    if not is_last:
        out_shape.append(jax.ShapeDtypeStruct((oxp_next, _N), jnp.float32))
        out_specs.append(pl.BlockSpec((oxp_next, _N), const))
    return pl.pallas_call(
        functools.partial(_node_body, oen, is_last),
        grid=(),
        in_specs=[
            pl.BlockSpec((nw, _N), const),
            pl.BlockSpec((_NW, wp, _N), lambda: (0, 0, 0)),
            pl.BlockSpec((oxn, nw), const),
            pl.BlockSpec((oxn, 1), const),
            pl.BlockSpec((oen, 1), const),
            pl.BlockSpec((oxp_next, nwn), const),
        ],
        out_specs=out_specs,
        out_shape=out_shape,
    )


def _prep_body(x_ref, w_ref, p_ref):
    p_ref[...] = jnp.dot(w_ref[...], x_ref[...],
                         preferred_element_type=jnp.float32)


def _make_prep(nw, oxp):
    const = lambda: (0, 0)
    return pl.pallas_call(
        _prep_body,
        grid=(),
        in_specs=[pl.BlockSpec((nw, _N), const),
                  pl.BlockSpec((oxp, nw), const)],
        out_specs=pl.BlockSpec((oxp, _N), const),
        out_shape=jax.ShapeDtypeStruct((oxp, _N), jnp.float32),
    )


@jax.jit
def kernel(x, edge_index, edge_attr, params):
    src = edge_index[0].astype(jnp.int32)
    dst = edge_index[1].astype(jnp.int32)
    xt = x.T
    h = edge_attr.T

    oxp1 = _pad8(_DIMS[0][2])
    p = _make_prep(x.shape[1], oxp1)(xt, _pad_rows(params["e1_wx"], oxp1))

    for k in range(5):
        e_nm, ixe, oxe, iee, oee = _DIMS[2 * k]
        n_nm, ixn, oxn, ien, oen = _DIMS[2 * k + 1]
        ewk = oxe + oee
        nwn = oxn + oen
        oxp = _pad8(oxe)
        wp = _pad8(oen)
        is_last = k == 4
        oxp_next = _pad8(_DIMS[2 * k + 2][2]) if not is_last else 8

        g = _make_gather(oxe, oxp)(p.reshape(-1), src, dst)
        h, s = _make_dense(iee, oxe, oxp, oee, ewk, wp)(
            g, h,
            params[e_nm + "_bx"].reshape(-1, 1),
            params[e_nm + "_we"],
            params[e_nm + "_be"].reshape(-1, 1),
            _pad_rows(params[n_nm + "_we"], wp),
        )
        aggp = _make_scatter(oen, wp)(
            s, src, jnp.zeros((wp * _N,), jnp.float32))
        wxe_next = (_pad_rows(params[_DIMS[2 * k + 2][0] + "_wx"], oxp_next)
                    if not is_last else jnp.zeros((8, nwn), jnp.float32))
        outs = _make_node(ixn, oxn, oen, wp, nwn, oxp_next, is_last)(
            xt, aggp.reshape(_NW, wp, _N),
            params[n_nm + "_wx"],
            params[n_nm + "_bx"].reshape(-1, 1),
            params[n_nm + "_be"].reshape(-1, 1),
            wxe_next,
        )
        if is_last:
            return outs[0].T
        xt, p = outs


# speedup vs baseline: 1.0332x; 1.0332x over previous
import functools

import jax
import jax.numpy as jnp
from jax import lax
from jax.experimental import pallas as pl
from jax.experimental.pallas import tpu as pltpu
from jax.experimental.pallas import tpu_sc as plsc

_DIMS = [
    ("e1", 5, 6, 1, 2), ("n1", 5, 6, 8, 10),
    ("e2", 16, 32, 8, 16), ("n2", 16, 32, 48, 24),
    ("e3", 56, 24, 48, 24), ("n3", 56, 24, 48, 24),
    ("e4", 48, 13, 48, 13), ("n4", 48, 13, 26, 8),
    ("e5", 21, 3, 26, 3), ("n5", 21, 3, 6, 3),
]

_N = 2048
_E = 65536
_NC, _NS = 2, 16
_NW = _NC * _NS
_EPW = _E // _NW
_QC = 512
_NQ = _EPW // _QC

_mesh = plsc.VectorSubcoreMesh(
    core_axis_name="c", subcore_axis_name="s", num_cores=_NC, num_subcores=_NS)


def _pad8(w):
    return ((w + 7) // 8) * 8


def _pad_rows(m, rows):
    if m.shape[0] == rows:
        return m
    return jnp.concatenate(
        [m, jnp.zeros((rows - m.shape[0],) + m.shape[1:], m.dtype)], axis=0)


def _make_gather(ox, oxp):
    out_t = jax.ShapeDtypeStruct((_NW, oxp, _EPW), jnp.float32)
    scratch = [
        pltpu.VMEM((oxp * _N,), jnp.float32),
        pltpu.VMEM((_EPW,), jnp.int32),
        pltpu.VMEM((_EPW,), jnp.int32),
        pltpu.VMEM((oxp, _QC), jnp.float32),
    ]

    @functools.partial(
        pl.kernel, out_type=out_t, mesh=_mesh, scratch_types=scratch,
        name=f"sc_gather_{ox}",
        compiler_params=pltpu.CompilerParams(needs_layout_passes=False))
    def gk(p_hbm, src_hbm, dst_hbm, g_hbm, tbl_v, is_v, id_v, out_v):
        wid = lax.axis_index("s") * _NC + lax.axis_index("c")
        ebase = wid * _EPW
        pltpu.sync_copy(p_hbm, tbl_v)
        pltpu.sync_copy(src_hbm.at[pl.ds(ebase, _EPW)], is_v)
        pltpu.sync_copy(dst_hbm.at[pl.ds(ebase, _EPW)], id_v)
        for q in range(_NQ):
            def body(i, _):
                o = q * _QC + i * 16
                i16 = is_v[pl.ds(o, 16)]
                j16 = id_v[pl.ds(o, 16)]
                for c in range(ox):
                    v = (plsc.load_gather(tbl_v, [i16 + c * _N])
                         + plsc.load_gather(tbl_v, [j16 + c * _N]))
                    out_v[c, pl.ds(i * 16, 16)] = v
                return ()

            lax.fori_loop(0, _QC // 16, body, ())
            pltpu.sync_copy(out_v, g_hbm.at[wid, :, pl.ds(q * _QC, _QC)])

    return gk


def _make_scatter(w, wp):
    out_t = jax.ShapeDtypeStruct((_NW, wp * _N), jnp.float32)
    scratch = [
        pltpu.VMEM((wp, _EPW), jnp.float32),
        pltpu.VMEM((_EPW,), jnp.int32),
        pltpu.VMEM((wp * _N,), jnp.float32),
    ]

    @functools.partial(
        pl.kernel, out_type=out_t, mesh=_mesh, scratch_types=scratch,
        name=f"sc_scatter_{w}",
        compiler_params=pltpu.CompilerParams(needs_layout_passes=False))
    def sk(s_hbm, src_hbm, zeros_hbm, agg_hbm, s_v, is_v, acc_v):
        wid = lax.axis_index("s") * _NC + lax.axis_index("c")
        ebase = wid * _EPW
        pltpu.sync_copy(s_hbm.at[wid], s_v)
        pltpu.sync_copy(src_hbm.at[pl.ds(ebase, _EPW)], is_v)
        pltpu.sync_copy(zeros_hbm, acc_v)

        def body(g, _):
            i16 = is_v[pl.ds(g * 16, 16)]
            for c in range(w):
                plsc.addupdate_scatter(acc_v, [i16 + c * _N],
                                       s_v[c, pl.ds(g * 16, 16)])
            return ()

        lax.fori_loop(0, _EPW // 16, body, ())
        pltpu.sync_copy(acc_v, agg_hbm.at[wid])

    return sk


def _dense_body(ox, _dense_w, gt_ref, h_ref, bxe_ref, wee_ref, bee_ref,
                wen_ref, hout_ref, s_ref):
    g = gt_ref[0][:ox] + bxe_ref[...]
    t = jnp.dot(wee_ref[...], h_ref[...].astype(jnp.float32),
                preferred_element_type=jnp.float32) + bee_ref[...]
    hk = jax.nn.relu(jnp.concatenate([g, t], axis=0))
    hout_ref[...] = hk.astype(jnp.bfloat16)
    s_ref[0] = jnp.dot(wen_ref[...], hk, preferred_element_type=jnp.float32)


def _make_dense(ew_prev, ox, oxp, oee, ewk, wp):
    const = lambda i: (0, 0)
    return pl.pallas_call(
        functools.partial(_dense_body, ox, wp),
        grid=(_NW,),
        in_specs=[
            pl.BlockSpec((1, oxp, _EPW), lambda i: (i, 0, 0)),
            pl.BlockSpec((ew_prev, _EPW), lambda i: (0, i)),
            pl.BlockSpec((ox, 1), const),
            pl.BlockSpec((oee, ew_prev), const),
            pl.BlockSpec((oee, 1), const),
            pl.BlockSpec((wp, ewk), const),
        ],
        out_specs=[
            pl.BlockSpec((ewk, _EPW), lambda i: (0, i)),
            pl.BlockSpec((1, wp, _EPW), lambda i: (i, 0, 0)),
        ],
        out_shape=[
            jax.ShapeDtypeStruct((ewk, _E), jnp.bfloat16),
            jax.ShapeDtypeStruct((_NW, wp, _EPW), jnp.float32),
        ],
    )


def _node_body(oen, is_last, x_ref, agg_ref, wxn_ref, bxn_ref, ben_ref,
               wxe_ref, x_out, p_out=None):
    agg = jnp.sum(agg_ref[...], axis=0)[:oen]
    xs = jnp.dot(wxn_ref[...], x_ref[...],
                 preferred_element_type=jnp.float32) + bxn_ref[...]
    xn = jax.nn.relu(jnp.concatenate([xs, agg + ben_ref[...]], axis=0))
    if is_last:
        state = jnp.sum(xn, axis=1, keepdims=True)
        x_out[...] = jnp.concatenate(
            [jnp.broadcast_to(state, xn.shape), xn], axis=0)
    else:
        x_out[...] = xn
        p_out[...] = jnp.dot(wxe_ref[...], xn,
                             preferred_element_type=jnp.float32)


def _make_node(nw, oxn, oen, wp, nwn, oxp_next, is_last):
    const = lambda: (0, 0)
    out_w = 2 * nwn if is_last else nwn
    out_shape = [jax.ShapeDtypeStruct((out_w, _N), jnp.float32)]
    out_specs = [pl.BlockSpec((out_w, _N), const)]
    if not is_last:
        out_shape.append(jax.ShapeDtypeStruct((oxp_next, _N), jnp.float32))
        out_specs.append(pl.BlockSpec((oxp_next, _N), const))
    return pl.pallas_call(
        functools.partial(_node_body, oen, is_last),
        grid=(),
        in_specs=[
            pl.BlockSpec((nw, _N), const),
            pl.BlockSpec((_NW, wp, _N), lambda: (0, 0, 0)),
            pl.BlockSpec((oxn, nw), const),
            pl.BlockSpec((oxn, 1), const),
            pl.BlockSpec((oen, 1), const),
            pl.BlockSpec((oxp_next, nwn), const),
        ],
        out_specs=out_specs,
        out_shape=out_shape,
    )


def _prep_body(x_ref, w_ref, p_ref):
    p_ref[...] = jnp.dot(w_ref[...], x_ref[...],
                         preferred_element_type=jnp.float32)


def _make_prep(nw, oxp):
    const = lambda: (0, 0)
    return pl.pallas_call(
        _prep_body,
        grid=(),
        in_specs=[pl.BlockSpec((nw, _N), const),
                  pl.BlockSpec((oxp, nw), const)],
        out_specs=pl.BlockSpec((oxp, _N), const),
        out_shape=jax.ShapeDtypeStruct((oxp, _N), jnp.float32),
    )


@jax.jit
def kernel(x, edge_index, edge_attr, params):
    src = edge_index[0].astype(jnp.int32)
    dst = edge_index[1].astype(jnp.int32)
    xt = x.T
    h = edge_attr.T

    oxp1 = _pad8(_DIMS[0][2])
    p = _make_prep(x.shape[1], oxp1)(xt, _pad_rows(params["e1_wx"], oxp1))

    for k in range(5):
        e_nm, ixe, oxe, iee, oee = _DIMS[2 * k]
        n_nm, ixn, oxn, ien, oen = _DIMS[2 * k + 1]
        ewk = oxe + oee
        nwn = oxn + oen
        oxp = _pad8(oxe)
        wp = _pad8(oen)
        is_last = k == 4
        oxp_next = _pad8(_DIMS[2 * k + 2][2]) if not is_last else 8

        g = _make_gather(oxe, oxp)(p.reshape(-1), src, dst)
        h, s = _make_dense(iee, oxe, oxp, oee, ewk, wp)(
            g, h,
            params[e_nm + "_bx"].reshape(-1, 1),
            params[e_nm + "_we"],
            params[e_nm + "_be"].reshape(-1, 1),
            _pad_rows(params[n_nm + "_we"], wp),
        )
        aggp = _make_scatter(oen, wp)(
            s, src, jnp.zeros((wp * _N,), jnp.float32))
        wxe_next = (_pad_rows(params[_DIMS[2 * k + 2][0] + "_wx"], oxp_next)
                    if not is_last else jnp.zeros((8, nwn), jnp.float32))
        outs = _make_node(ixn, oxn, oen, wp, nwn, oxp_next, is_last)(
            xt, aggp.reshape(_NW, wp, _N),
            params[n_nm + "_wx"],
            params[n_nm + "_bx"].reshape(-1, 1),
            params[n_nm + "_be"].reshape(-1, 1),
            wxe_next,
        )
        if is_last:
            return outs[0].T
        xt, p = outs


# speedup vs baseline: 1.0467x; 1.0131x over previous
import functools

import jax
import jax.numpy as jnp
from jax import lax
from jax.experimental import pallas as pl
from jax.experimental.pallas import tpu as pltpu
from jax.experimental.pallas import tpu_sc as plsc

_DIMS = [
    ("e1", 5, 6, 1, 2), ("n1", 5, 6, 8, 10),
    ("e2", 16, 32, 8, 16), ("n2", 16, 32, 48, 24),
    ("e3", 56, 24, 48, 24), ("n3", 56, 24, 48, 24),
    ("e4", 48, 13, 48, 13), ("n4", 48, 13, 26, 8),
    ("e5", 21, 3, 26, 3), ("n5", 21, 3, 6, 3),
]

_N = 2048
_E = 65536
_NC, _NS = 2, 16
_NW = _NC * _NS
_EPW = _E // _NW
_QC = 512
_NQ = _EPW // _QC

_mesh = plsc.VectorSubcoreMesh(
    core_axis_name="c", subcore_axis_name="s", num_cores=_NC, num_subcores=_NS)


def _pad8(w):
    return ((w + 7) // 8) * 8


def _pad_rows(m, rows):
    if m.shape[0] == rows:
        return m
    return jnp.concatenate(
        [m, jnp.zeros((rows - m.shape[0],) + m.shape[1:], m.dtype)], axis=0)


def _make_gather(ox, oxp):
    out_t = jax.ShapeDtypeStruct((_NW, oxp, _EPW), jnp.float32)
    scratch = [
        pltpu.VMEM((oxp * _N,), jnp.float32),
        pltpu.VMEM((_EPW,), jnp.int32),
        pltpu.VMEM((_EPW,), jnp.int32),
        pltpu.VMEM((oxp, _QC), jnp.float32),
        pltpu.SemaphoreType.DMA,
        pltpu.SemaphoreType.DMA,
        pltpu.SemaphoreType.DMA,
    ]

    @functools.partial(
        pl.kernel, out_type=out_t, mesh=_mesh, scratch_types=scratch,
        name=f"sc_gather_{ox}",
        compiler_params=pltpu.CompilerParams(needs_layout_passes=False))
    def gk(p_hbm, src_hbm, dst_hbm, g_hbm, tbl_v, is_v, id_v, out_v,
           m0, m1, m2):
        wid = lax.axis_index("s") * _NC + lax.axis_index("c")
        ebase = wid * _EPW
        d0 = pltpu.async_copy(p_hbm, tbl_v, m0)
        d1 = pltpu.async_copy(src_hbm.at[pl.ds(ebase, _EPW)], is_v, m1)
        d2 = pltpu.async_copy(dst_hbm.at[pl.ds(ebase, _EPW)], id_v, m2)
        d0.wait()
        d1.wait()
        d2.wait()
        for q in range(_NQ):
            def body(i, _):
                o = q * _QC + i * 16
                i16 = is_v[pl.ds(o, 16)]
                j16 = id_v[pl.ds(o, 16)]
                for c in range(ox):
                    v = (plsc.load_gather(tbl_v, [i16 + c * _N])
                         + plsc.load_gather(tbl_v, [j16 + c * _N]))
                    out_v[c, pl.ds(i * 16, 16)] = v
                return ()

            lax.fori_loop(0, _QC // 16, body, ())
            pltpu.sync_copy(out_v, g_hbm.at[wid, :, pl.ds(q * _QC, _QC)])

    return gk


def _make_scatter(w, wp):
    out_t = jax.ShapeDtypeStruct((_NW, wp * _N), jnp.float32)
    scratch = [
        pltpu.VMEM((wp, _EPW), jnp.float32),
        pltpu.VMEM((_EPW,), jnp.int32),
        pltpu.VMEM((wp * _N,), jnp.float32),
        pltpu.SemaphoreType.DMA,
        pltpu.SemaphoreType.DMA,
        pltpu.SemaphoreType.DMA,
    ]

    @functools.partial(
        pl.kernel, out_type=out_t, mesh=_mesh, scratch_types=scratch,
        name=f"sc_scatter_{w}",
        compiler_params=pltpu.CompilerParams(needs_layout_passes=False))
    def sk(s_hbm, src_hbm, zeros_hbm, agg_hbm, s_v, is_v, acc_v, m0, m1, m2):
        wid = lax.axis_index("s") * _NC + lax.axis_index("c")
        ebase = wid * _EPW
        d0 = pltpu.async_copy(s_hbm.at[wid], s_v, m0)
        d1 = pltpu.async_copy(src_hbm.at[pl.ds(ebase, _EPW)], is_v, m1)
        d2 = pltpu.async_copy(zeros_hbm, acc_v, m2)
        d0.wait()
        d1.wait()
        d2.wait()

        def body(g, _):
            i16 = is_v[pl.ds(g * 16, 16)]
            for c in range(w):
                plsc.addupdate_scatter(acc_v, [i16 + c * _N],
                                       s_v[c, pl.ds(g * 16, 16)])
            return ()

        lax.fori_loop(0, _EPW // 16, body, ())
        pltpu.sync_copy(acc_v, agg_hbm.at[wid])

    return sk


def _dense_body(ox, _dense_w, gt_ref, h_ref, bxe_ref, wee_ref, bee_ref,
                wen_ref, hout_ref, s_ref):
    g = gt_ref[0][:ox] + bxe_ref[...]
    t = jnp.dot(wee_ref[...], h_ref[...].astype(jnp.float32),
                preferred_element_type=jnp.float32) + bee_ref[...]
    hk = jax.nn.relu(jnp.concatenate([g, t], axis=0))
    hout_ref[...] = hk.astype(jnp.bfloat16)
    s_ref[0] = jnp.dot(wen_ref[...], hk, preferred_element_type=jnp.float32)


def _make_dense(ew_prev, ox, oxp, oee, ewk, wp):
    const = lambda i: (0, 0)
    return pl.pallas_call(
        functools.partial(_dense_body, ox, wp),
        grid=(_NW,),
        in_specs=[
            pl.BlockSpec((1, oxp, _EPW), lambda i: (i, 0, 0)),
            pl.BlockSpec((ew_prev, _EPW), lambda i: (0, i)),
            pl.BlockSpec((ox, 1), const),
            pl.BlockSpec((oee, ew_prev), const),
            pl.BlockSpec((oee, 1), const),
            pl.BlockSpec((wp, ewk), const),
        ],
        out_specs=[
            pl.BlockSpec((ewk, _EPW), lambda i: (0, i)),
            pl.BlockSpec((1, wp, _EPW), lambda i: (i, 0, 0)),
        ],
        out_shape=[
            jax.ShapeDtypeStruct((ewk, _E), jnp.bfloat16),
            jax.ShapeDtypeStruct((_NW, wp, _EPW), jnp.float32),
        ],
    )


def _node_body(oen, is_last, x_ref, agg_ref, wxn_ref, bxn_ref, ben_ref,
               wxe_ref, x_out, p_out=None):
    agg = jnp.sum(agg_ref[...], axis=0)[:oen]
    xs = jnp.dot(wxn_ref[...], x_ref[...],
                 preferred_element_type=jnp.float32) + bxn_ref[...]
    xn = jax.nn.relu(jnp.concatenate([xs, agg + ben_ref[...]], axis=0))
    if is_last:
        state = jnp.sum(xn, axis=1, keepdims=True)
        x_out[...] = jnp.concatenate(
            [jnp.broadcast_to(state, xn.shape), xn], axis=0)
    else:
        x_out[...] = xn
        p_out[...] = jnp.dot(wxe_ref[...], xn,
                             preferred_element_type=jnp.float32)


def _make_node(nw, oxn, oen, wp, nwn, oxp_next, is_last):
    const = lambda: (0, 0)
    out_w = 2 * nwn if is_last else nwn
    out_shape = [jax.ShapeDtypeStruct((out_w, _N), jnp.float32)]
    out_specs = [pl.BlockSpec((out_w, _N), const)]
    if not is_last:
        out_shape.append(jax.ShapeDtypeStruct((oxp_next, _N), jnp.float32))
        out_specs.append(pl.BlockSpec((oxp_next, _N), const))
    return pl.pallas_call(
        functools.partial(_node_body, oen, is_last),
        grid=(),
        in_specs=[
            pl.BlockSpec((nw, _N), const),
            pl.BlockSpec((_NW, wp, _N), lambda: (0, 0, 0)),
            pl.BlockSpec((oxn, nw), const),
            pl.BlockSpec((oxn, 1), const),
            pl.BlockSpec((oen, 1), const),
            pl.BlockSpec((oxp_next, nwn), const),
        ],
        out_specs=out_specs,
        out_shape=out_shape,
    )


def _prep_body(x_ref, w_ref, p_ref):
    p_ref[...] = jnp.dot(w_ref[...], x_ref[...],
                         preferred_element_type=jnp.float32)


def _make_prep(nw, oxp):
    const = lambda: (0, 0)
    return pl.pallas_call(
        _prep_body,
        grid=(),
        in_specs=[pl.BlockSpec((nw, _N), const),
                  pl.BlockSpec((oxp, nw), const)],
        out_specs=pl.BlockSpec((oxp, _N), const),
        out_shape=jax.ShapeDtypeStruct((oxp, _N), jnp.float32),
    )


@jax.jit
def kernel(x, edge_index, edge_attr, params):
    src = edge_index[0].astype(jnp.int32)
    dst = edge_index[1].astype(jnp.int32)
    xt = x.T
    h = edge_attr.T

    oxp1 = _pad8(_DIMS[0][2])
    p = _make_prep(x.shape[1], oxp1)(xt, _pad_rows(params["e1_wx"], oxp1))

    for k in range(5):
        e_nm, ixe, oxe, iee, oee = _DIMS[2 * k]
        n_nm, ixn, oxn, ien, oen = _DIMS[2 * k + 1]
        ewk = oxe + oee
        nwn = oxn + oen
        oxp = _pad8(oxe)
        wp = _pad8(oen)
        is_last = k == 4
        oxp_next = _pad8(_DIMS[2 * k + 2][2]) if not is_last else 8

        g = _make_gather(oxe, oxp)(p.reshape(-1), src, dst)
        h, s = _make_dense(iee, oxe, oxp, oee, ewk, wp)(
            g, h,
            params[e_nm + "_bx"].reshape(-1, 1),
            params[e_nm + "_we"],
            params[e_nm + "_be"].reshape(-1, 1),
            _pad_rows(params[n_nm + "_we"], wp),
        )
        aggp = _make_scatter(oen, wp)(
            s, src, jnp.zeros((wp * _N,), jnp.float32))
        wxe_next = (_pad_rows(params[_DIMS[2 * k + 2][0] + "_wx"], oxp_next)
                    if not is_last else jnp.zeros((8, nwn), jnp.float32))
        outs = _make_node(ixn, oxn, oen, wp, nwn, oxp_next, is_last)(
            xt, aggp.reshape(_NW, wp, _N),
            params[n_nm + "_wx"],
            params[n_nm + "_bx"].reshape(-1, 1),
            params[n_nm + "_be"].reshape(-1, 1),
            wxe_next,
        )
        if is_last:
            return outs[0].T
        xt, p = outs
